# baseline (speedup 1.0000x reference)
import functools

import jax
import jax.numpy as jnp
from jax import lax
from jax.experimental import pallas as pl
from jax.experimental.pallas import tpu as pltpu
from jax.experimental.pallas import tpu_sc as plsc

_NC = 2
_NS = 16
_L = 16
_NW = _NC * _NS

_B = 128


def _xlen(F):
    return ((_B * F + 7 + _L + 7) // 8) * 8


def _sc_body(x_hbm, tab_hbm, out_hbm, tab_v, out_vs, xb_vs, sems, *,
             F, V, D, N):
    wid = lax.axis_index("s") * _NC + lax.axis_index("c")
    K = D // 2
    rows_w = N // _NW
    nb = -(-rows_w // _B)
    xlen = _xlen(F)
    sx, so = sems[:2], sems[2:]
    pltpu.sync_copy(tab_hbm, tab_v)

    def xslice(j):
        base = wid * rows_w + jnp.minimum(j * _B, rows_w - _B)
        xstart = pl.multiple_of((base * F >> 3) << 3, 8)
        return base, xstart

    for p in range(2):
        _, xstart = xslice(p)
        pltpu.async_copy(x_hbm.at[pl.ds(xstart, xlen)], xb_vs[p], sx[p])

    def batch_body(j, carry):
        def run(p):
            base, xstart = xslice(j)
            delta = base * F - xstart
            xb_v, out_v = xb_vs[p], out_vs[p]
            pltpu.make_async_copy(x_hbm.at[pl.ds(xstart, xlen)], xb_v,
                                  sx[p]).wait()

            @pl.when(j >= 2)
            def _():
                pltpu.make_async_copy(
                    out_v, out_hbm.at[pl.ds(0, _B * D)], so[p]).wait()

            def group_body(gi, c2):
                goff = delta + gi * (_L * F)
                for b in range(_L):
                    xv = xb_v[pl.ds(goff + b * F, _L)]
                    rows = [(xv[i] + i * V) * K for i in range(F)]
                    out_row = gi * (_L * D) + b * D
                    for c in range(K // _L):
                        g = [plsc.bitcast(
                                tab_v[pl.ds(rows[i] + c * _L, _L)],
                                jnp.bfloat16)
                             for i in range(F)]
                        while len(g) > 1:
                            g = [g[j2] + g[j2 + 1]
                                 for j2 in range(0, len(g) - 1, 2)] \
                                + ([g[-1]] if len(g) % 2 else [])
                        lo, hi = plsc.unpack(
                            g[0], format=plsc.PackFormat.INTERLEAVED)
                        out_v[pl.ds(out_row + c * 2 * _L, _L)] = lo
                        out_v[pl.ds(out_row + c * 2 * _L + _L, _L)] = hi
                return c2

            lax.fori_loop(0, _B // _L, group_body, 0, unroll=False)
            pltpu.async_copy(
                out_v,
                out_hbm.at[pl.ds(pl.multiple_of(base * D, 8), _B * D)],
                so[p])

            @pl.when(j + 2 < nb)
            def _():
                _, xs2 = xslice(j + 2)
                pltpu.async_copy(x_hbm.at[pl.ds(xs2, xlen)], xb_v, sx[p])

        @pl.when(j % 2 == 0)
        def _():
            run(0)

        @pl.when(j % 2 == 1)
        def _():
            run(1)

        return carry

    lax.fori_loop(0, nb, batch_body, 0, unroll=False)
    for p in range(2):
        pltpu.make_async_copy(out_vs[p], out_hbm.at[pl.ds(0, _B * D)],
                              so[p]).wait()


def kernel(x, tables):
    N, F = x.shape
    _, V, D = tables.shape
    K = D // 2

    x_flat = jnp.pad(x.reshape(N * F), (0, _xlen(F)))
    tab_pairs = lax.bitcast_convert_type(
        tables.astype(jnp.bfloat16).reshape(F * V, D // 32, 2, _L)
        .transpose(0, 1, 3, 2).reshape(F * V * K, 2), jnp.int32)

    mesh = plsc.VectorSubcoreMesh(core_axis_name="c", subcore_axis_name="s")
    body = functools.partial(_sc_body, F=F, V=V, D=D, N=N)
    out = pl.kernel(
        body,
        out_type=jax.ShapeDtypeStruct((N * D,), jnp.float32),
        mesh=mesh,
        scratch_types=[
            pltpu.VMEM((F * V * K,), jnp.int32),
            [pltpu.VMEM((_B * D,), jnp.float32)] * 2,
            [pltpu.VMEM((_xlen(F),), jnp.int32)] * 2,
            [pltpu.SemaphoreType.DMA] * 4,
        ],
        compiler_params=pltpu.CompilerParams(needs_layout_passes=False),
    )(x_flat, tab_pairs)
    return out.reshape(N, D)


# speedup vs baseline: 1.2537x; 1.2537x over previous
import functools

import jax
import jax.numpy as jnp
from jax import lax
from jax.experimental import pallas as pl
from jax.experimental.pallas import tpu as pltpu
from jax.experimental.pallas import tpu_sc as plsc

_NC = 2
_NS = 16
_L = 16
_NW = _NC * _NS

_B = 128


def _xlen(F):
    return ((_B * F + 7 + _L + 7) // 8) * 8


def _sc_body(x_hbm, tab_hbm, out_hbm, tab_v, out_v, xb_v, *, F, V, D, N):
    wid = lax.axis_index("s") * _NC + lax.axis_index("c")
    K = D // 2
    rows_w = N // _NW
    nb = -(-rows_w // _B)
    xlen = _xlen(F)
    pltpu.sync_copy(tab_hbm, tab_v)

    iota = lax.iota(jnp.int32, _L)
    offv = jnp.where(iota < F, iota * V, 0) * K
    splat_idx = [jnp.full((_L, 1), i, jnp.int32) for i in range(F)]
    gdims = lax.GatherDimensionNumbers(
        offset_dims=(), collapsed_slice_dims=(0,), start_index_map=(0,))
    chunk_iota = [iota + c * _L for c in range(D // (2 * _L))]

    def batch_body(j, carry):
        base = wid * rows_w + jnp.minimum(j * _B, rows_w - _B)
        xoff = base * F
        xstart = pl.multiple_of((xoff >> 3) << 3, 8)
        delta = xoff - xstart
        pltpu.sync_copy(x_hbm.at[pl.ds(xstart, xlen)], xb_v)

        @plsc.parallel_loop(0, _B, step=1, unroll=2)
        def row_body(b):
            xv = xb_v[pl.ds(delta + b * F, _L)]
            bases = xv * K + offv
            addr = [lax.gather(
                        bases, splat_idx[i], gdims, slice_sizes=(1,),
                        mode=lax.GatherScatterMode.PROMISE_IN_BOUNDS)
                    for i in range(F)]
            out_row = b * D
            for c in range(K // _L):
                g = [plsc.bitcast(
                        plsc.load_gather(tab_v, [addr[i] + chunk_iota[c]]),
                        jnp.bfloat16)
                     for i in range(F)]
                while len(g) > 1:
                    g = [g[j2] + g[j2 + 1]
                         for j2 in range(0, len(g) - 1, 2)] \
                        + ([g[-1]] if len(g) % 2 else [])
                lo, hi = plsc.unpack(
                    g[0], format=plsc.PackFormat.INTERLEAVED)
                out_v[pl.ds(out_row + c * 2 * _L, _L)] = lo
                out_v[pl.ds(out_row + c * 2 * _L + _L, _L)] = hi
        pltpu.sync_copy(
            out_v, out_hbm.at[pl.ds(pl.multiple_of(base * D, 8), _B * D)])
        return carry

    lax.fori_loop(0, nb, batch_body, 0, unroll=False)


def kernel(x, tables):
    N, F = x.shape
    _, V, D = tables.shape
    K = D // 2

    x_flat = jnp.pad(x.reshape(N * F), (0, _xlen(F)))
    tab_pairs = lax.bitcast_convert_type(
        tables.astype(jnp.bfloat16).reshape(F * V, D // 32, 2, _L)
        .transpose(0, 1, 3, 2).reshape(F * V * K, 2), jnp.int32)

    mesh = plsc.VectorSubcoreMesh(core_axis_name="c", subcore_axis_name="s")
    body = functools.partial(_sc_body, F=F, V=V, D=D, N=N)
    out = pl.kernel(
        body,
        out_type=jax.ShapeDtypeStruct((N * D,), jnp.float32),
        mesh=mesh,
        scratch_types=[
            pltpu.VMEM((F * V * K,), jnp.int32),
            pltpu.VMEM((_B * D,), jnp.float32),
            pltpu.VMEM((_xlen(F),), jnp.int32),
        ],
        compiler_params=pltpu.CompilerParams(needs_layout_passes=False),
    )(x_flat, tab_pairs)
    return out.reshape(N, D)


# speedup vs baseline: 1.4757x; 1.1770x over previous
import functools

import jax
import jax.numpy as jnp
from jax import lax
from jax.experimental import pallas as pl
from jax.experimental.pallas import tpu as pltpu
from jax.experimental.pallas import tpu_sc as plsc

_NC = 2
_NS = 16
_L = 16
_NW = _NC * _NS

_B = 128


def _xlen(F):
    return ((_B * F + 7 + _L + 7) // 8) * 8


def _sc_body(x_hbm, tab_hbm, out_hbm, tab_v, out_vs, xb_vs, sems, *,
             F, V, D, N):
    wid = lax.axis_index("s") * _NC + lax.axis_index("c")
    K = D // 2
    rows_w = N // _NW
    nb = -(-rows_w // _B)
    nslots = nb + (nb % 2)
    xlen = _xlen(F)
    sx, so = sems[:2], sems[2:]
    pltpu.sync_copy(tab_hbm, tab_v)

    iota = lax.iota(jnp.int32, _L)
    offv = jnp.where(iota < F, iota * V, 0) * K
    splat_idx = [jnp.full((_L, 1), i, jnp.int32) for i in range(F)]
    gdims = lax.GatherDimensionNumbers(
        offset_dims=(), collapsed_slice_dims=(0,), start_index_map=(0,))
    chunk_iota = [iota + c * _L for c in range(D // (2 * _L))]

    def xwin(k):
        base = wid * rows_w + jnp.minimum(k * _B, rows_w - _B)
        xstart = pl.multiple_of((base * F >> 3) << 3, 8)
        return base, xstart

    for p in range(2):
        _, xstart = xwin(p)
        pltpu.async_copy(x_hbm.at[pl.ds(xstart, xlen)], xb_vs[p], sx[p])

    def batch_pair(j2, carry):
        for p in range(2):
            k = 2 * j2 + p
            base, xstart = xwin(k)
            delta = base * F - xstart
            xb_v, out_v = xb_vs[p], out_vs[p]
            pltpu.make_async_copy(x_hbm.at[pl.ds(xstart, xlen)], xb_v,
                                  sx[p]).wait()

            @pl.when(j2 >= 1)
            def _():
                pltpu.make_async_copy(
                    out_v, out_hbm.at[pl.ds(0, _B * D)], so[p]).wait()

            @plsc.parallel_loop(0, _B, step=1, unroll=2)
            def row_body(b):
                xv = xb_v[pl.ds(delta + b * F, _L)]
                bases = xv * K + offv
                addr = [lax.gather(
                            bases, splat_idx[i], gdims, slice_sizes=(1,),
                            mode=lax.GatherScatterMode.PROMISE_IN_BOUNDS)
                        for i in range(F)]
                out_row = b * D
                for c in range(K // _L):
                    g = [plsc.bitcast(
                            plsc.load_gather(tab_v,
                                             [addr[i] + chunk_iota[c]]),
                            jnp.bfloat16)
                         for i in range(F)]
                    while len(g) > 1:
                        g = [g[j3] + g[j3 + 1]
                             for j3 in range(0, len(g) - 1, 2)] \
                            + ([g[-1]] if len(g) % 2 else [])
                    lo, hi = plsc.unpack(
                        g[0], format=plsc.PackFormat.INTERLEAVED)
                    out_v[pl.ds(out_row + c * 2 * _L, _L)] = lo
                    out_v[pl.ds(out_row + c * 2 * _L + _L, _L)] = hi

            pltpu.async_copy(
                out_v,
                out_hbm.at[pl.ds(pl.multiple_of(base * D, 8), _B * D)],
                so[p])

            @pl.when(k + 2 < nslots)
            def _():
                _, xs2 = xwin(k + 2)
                pltpu.async_copy(x_hbm.at[pl.ds(xs2, xlen)], xb_v, sx[p])
        return carry

    lax.fori_loop(0, nslots // 2, batch_pair, 0, unroll=False)
    for p in range(2):
        pltpu.make_async_copy(out_vs[p], out_hbm.at[pl.ds(0, _B * D)],
                              so[p]).wait()


def kernel(x, tables):
    N, F = x.shape
    _, V, D = tables.shape
    K = D // 2

    x_flat = jnp.pad(x.reshape(N * F), (0, _xlen(F)))
    tab_pairs = lax.bitcast_convert_type(
        tables.astype(jnp.bfloat16).reshape(F * V, D // 32, 2, _L)
        .transpose(0, 1, 3, 2).reshape(F * V * K, 2), jnp.int32)

    mesh = plsc.VectorSubcoreMesh(core_axis_name="c", subcore_axis_name="s")
    body = functools.partial(_sc_body, F=F, V=V, D=D, N=N)
    out = pl.kernel(
        body,
        out_type=jax.ShapeDtypeStruct((N * D,), jnp.float32),
        mesh=mesh,
        scratch_types=[
            pltpu.VMEM((F * V * K,), jnp.int32),
            [pltpu.VMEM((_B * D,), jnp.float32)] * 2,
            [pltpu.VMEM((_xlen(F),), jnp.int32)] * 2,
            [pltpu.SemaphoreType.DMA] * 4,
        ],
        compiler_params=pltpu.CompilerParams(needs_layout_passes=False),
    )(x_flat, tab_pairs)
    return out.reshape(N, D)
